# baseline (speedup 1.0000x reference)
import jax
import jax.numpy as jnp
from jax import lax
from jax.experimental import pallas as pl
from jax.experimental.pallas import tpu as pltpu
from jax.experimental.pallas import tpu_sc as plsc

VOCAB = 100
HIDDEN = 1024
BATCH = 4096
SEQ = 20

NC, NS, L = 2, 16, 16
NW = NC * NS
NBLK = NW // 2
BPB = BATCH // NBLK
HH = HIDDEN // 2
NJ = HH // L

_mesh = plsc.VectorSubcoreMesh(core_axis_name="c", subcore_axis_name="s")

_DNUMS = lax.GatherDimensionNumbers(
    offset_dims=(), collapsed_slice_dims=(0,), start_index_map=(0,)
)


def _bcast_lane(vec, lane):
    return lax.gather(
        vec,
        jnp.full((L, 1), lane, jnp.int32),
        _DNUMS,
        (1,),
        mode=lax.GatherScatterMode.PROMISE_IN_BOUNDS,
    )


@jax.jit
def _sc_gather(table, idx):
    @pl.kernel(
        out_type=jax.ShapeDtypeStruct((BATCH, SEQ, HIDDEN), jnp.float32),
        mesh=_mesh,
        scratch_types=[
            pltpu.VMEM((VOCAB, HH), jnp.float32),
            pltpu.VMEM((BPB * SEQ,), jnp.int32),
            pltpu.VMEM((SEQ, HH), jnp.float32),
            pltpu.VMEM((SEQ, HH), jnp.float32),
            pltpu.SemaphoreType.DMA,
            pltpu.SemaphoreType.DMA,
        ],
        compiler_params=pltpu.CompilerParams(needs_layout_passes=False),
    )
    def k(table_hbm, idx_hbm, out_hbm, table_v, idx_v, b0, b1, w0, w1):
        wid = lax.axis_index("s") * NC + lax.axis_index("c")
        blk = wid // 2
        hoff = (wid % 2) * HH
        base_b = blk * BPB
        pltpu.sync_copy(
            table_hbm.at[pl.ds(0, VOCAB), pl.ds(hoff, HH)], table_v
        )
        pltpu.sync_copy(idx_hbm.at[blk], idx_v)
        bufs = (b0, b1)
        wsems = (w0, w1)
        iota = lax.iota(jnp.int32, L)

        def do_batch(bb, s):
            @pl.when(bb >= 2)
            def _():
                pltpu.make_async_copy(
                    bufs[s],
                    out_hbm.at[base_b, pl.ds(0, SEQ), pl.ds(hoff, HH)],
                    wsems[s],
                ).wait()
            va = idx_v[pl.ds(bb * SEQ, L)]
            vb = idx_v[pl.ds(bb * SEQ + (SEQ - L), L)]
            for rr in range(SEQ):
                if rr < L:
                    rsplat = _bcast_lane(va, rr)
                else:
                    rsplat = _bcast_lane(vb, rr - (SEQ - L))
                colv = iota
                for jg in range(NJ // 16):
                    vs = []
                    for u in range(16):
                        vs.append(plsc.load_gather(table_v, [rsplat, colv]))
                        colv = colv + L
                    for u in range(16):
                        bufs[s][rr, pl.ds((jg * 16 + u) * L, L)] = vs[u]
            pltpu.async_copy(
                bufs[s],
                out_hbm.at[base_b + bb, pl.ds(0, SEQ), pl.ds(hoff, HH)],
                wsems[s],
            )

        def pair(i, carry):
            do_batch(i * 2, 0)
            do_batch(i * 2 + 1, 1)
            return carry

        lax.fori_loop(0, BPB // 2, pair, 0)
        for s in range(2):
            pltpu.make_async_copy(
                bufs[s],
                out_hbm.at[base_b, pl.ds(0, SEQ), pl.ds(hoff, HH)],
                wsems[s],
            ).wait()

    return k(table, idx)


def kernel(indices, word_embeddings):
    idx = indices.reshape(NBLK, BPB * SEQ)
    return _sc_gather(word_embeddings, idx)


# speedup vs baseline: 1.0315x; 1.0315x over previous
import jax
import jax.numpy as jnp
from jax import lax
from jax.experimental import pallas as pl
from jax.experimental.pallas import tpu as pltpu
from jax.experimental.pallas import tpu_sc as plsc

VOCAB = 100
HIDDEN = 1024
BATCH = 4096
SEQ = 20

NC, NS, L = 2, 16, 16
NW = NC * NS
NBLK = NW // 2
BPB = BATCH // NBLK
HH = HIDDEN // 2
NJ = HH // L

_mesh = plsc.VectorSubcoreMesh(core_axis_name="c", subcore_axis_name="s")

_DNUMS = lax.GatherDimensionNumbers(
    offset_dims=(), collapsed_slice_dims=(0,), start_index_map=(0,)
)


def _bcast_lane(vec, lane):
    return lax.gather(
        vec,
        jnp.full((L, 1), lane, jnp.int32),
        _DNUMS,
        (1,),
        mode=lax.GatherScatterMode.PROMISE_IN_BOUNDS,
    )


@jax.jit
def _sc_gather(table, idx):
    @pl.kernel(
        out_type=jax.ShapeDtypeStruct((BATCH, SEQ, HIDDEN), jnp.float32),
        mesh=_mesh,
        scratch_types=[
            pltpu.VMEM((VOCAB, HH), jnp.float32),
            pltpu.VMEM((BPB * SEQ,), jnp.int32),
            pltpu.VMEM((SEQ, HH), jnp.float32),
            pltpu.VMEM((SEQ, HH), jnp.float32),
            pltpu.SemaphoreType.DMA,
            pltpu.SemaphoreType.DMA,
        ],
        compiler_params=pltpu.CompilerParams(needs_layout_passes=False),
    )
    def k(table_hbm, idx_hbm, out_hbm, table_v, idx_v, b0, b1, w0, w1):
        wid = lax.axis_index("s") * NC + lax.axis_index("c")
        blk = wid // 2
        hoff = (wid % 2) * HH
        base_b = blk * BPB
        pltpu.sync_copy(
            table_hbm.at[pl.ds(0, VOCAB), pl.ds(hoff, HH)], table_v
        )
        pltpu.sync_copy(idx_hbm.at[blk], idx_v)
        bufs = (b0, b1)
        wsems = (w0, w1)
        iota = lax.iota(jnp.int32, L)

        def do_batch(bb, s):
            @pl.when(bb >= 2)
            def _():
                pltpu.make_async_copy(
                    bufs[s],
                    out_hbm.at[base_b, pl.ds(0, SEQ), pl.ds(hoff, HH)],
                    wsems[s],
                ).wait()
            va = idx_v[pl.ds(bb * SEQ, L)]
            vb = idx_v[pl.ds(bb * SEQ + (SEQ - L), L)]
            for rr in range(SEQ):
                if rr < L:
                    rsplat = _bcast_lane(va, rr)
                else:
                    rsplat = _bcast_lane(vb, rr - (SEQ - L))
                colv = iota
                for jg in range(NJ // 8):
                    vs = []
                    for u in range(8):
                        vs.append(plsc.load_gather(table_v, [rsplat, colv]))
                        colv = colv + L
                    for u in range(8):
                        bufs[s][rr, pl.ds((jg * 8 + u) * L, L)] = vs[u]
            pltpu.async_copy(
                bufs[s],
                out_hbm.at[base_b + bb, pl.ds(0, SEQ), pl.ds(hoff, HH)],
                wsems[s],
            )

        def pair(i, carry):
            do_batch(i * 2, 0)
            do_batch(i * 2 + 1, 1)
            return carry

        lax.fori_loop(0, BPB // 2, pair, 0)
        for s in range(2):
            pltpu.make_async_copy(
                bufs[s],
                out_hbm.at[base_b, pl.ds(0, SEQ), pl.ds(hoff, HH)],
                wsems[s],
            ).wait()

    return k(table, idx)


def kernel(indices, word_embeddings):
    idx = indices.reshape(NBLK, BPB * SEQ)
    return _sc_gather(word_embeddings, idx)
